# baseline (speedup 1.0000x reference)
import functools

import jax
import jax.numpy as jnp
from jax import lax
from jax.experimental import pallas as pl
from jax.experimental.pallas import tpu as pltpu
from jax.experimental.pallas import tpu_sc as plsc

_NC = 2
_NS = 16
_NW = _NC * _NS
_CHUNK = 128
_K = 10


def _make_gather(n_chunks: int, emb_dim: int):
    n_groups = n_chunks // _K
    assert n_chunks % _K == 0 and n_groups % 2 == 0 and n_groups >= 4
    mesh = plsc.VectorSubcoreMesh(core_axis_name="c", subcore_axis_name="s")

    @functools.partial(
        pl.kernel,
        out_type=jax.ShapeDtypeStruct((_NW, n_chunks, _CHUNK, emb_dim),
                                      jnp.float32),
        mesh=mesh,
        compiler_params=pltpu.CompilerParams(use_tc_tiling_on_sc=False),
        scratch_types=[
            pltpu.VMEM((n_chunks, _CHUNK), jnp.int32),
            pltpu.VMEM((2 * _K, _CHUNK, emb_dim), jnp.float32),
            pltpu.SemaphoreType.DMA,
            pltpu.SemaphoreType.DMA,
            pltpu.SemaphoreType.DMA,
            pltpu.SemaphoreType.DMA,
        ],
    )
    def gather_kernel(ids_hbm, table_hbm, out_hbm, idx_v, rows_v,
                      gsem0, gsem1, wsem0, wsem1):
        wid = lax.axis_index("s") * _NC + lax.axis_index("c")
        gsems = (gsem0, gsem1)
        wsems = (wsem0, wsem1)

        pltpu.sync_copy(ids_hbm.at[wid], idx_v)

        for b in range(_K):
            pltpu.async_copy(table_hbm.at[idx_v.at[b]], rows_v.at[b], gsem0)

        @pl.loop(0, n_groups, step=2)
        def _(g0):
            for h in (0, 1):
                g = g0 + h
                my = h * _K
                other = (1 - h) * _K
                @pl.when(jnp.logical_and(g >= 1, g + 1 < n_groups))
                def _():
                    for b in range(_K):
                        pltpu.make_async_copy(
                            rows_v.at[other + b],
                            out_hbm.at[wid, (g - 1) * _K + b],
                            wsems[1 - h]).wait()

                @pl.when(g + 1 < n_groups)
                def _():
                    for b in range(_K):
                        pltpu.async_copy(
                            table_hbm.at[idx_v.at[(g + 1) * _K + b]],
                            rows_v.at[other + b], gsems[1 - h])

                for b in range(_K):
                    pltpu.make_async_copy(
                        table_hbm.at[idx_v.at[g * _K + b]],
                        rows_v.at[my + b], gsems[h]).wait()
                for b in range(_K):
                    pltpu.async_copy(rows_v.at[my + b],
                                     out_hbm.at[wid, g * _K + b], wsems[h])

        for b in range(_K):
            pltpu.make_async_copy(rows_v.at[b],
                                  out_hbm.at[wid, (n_groups - 2) * _K + b],
                                  wsem0).wait()
        for b in range(_K):
            pltpu.make_async_copy(rows_v.at[_K + b],
                                  out_hbm.at[wid, (n_groups - 1) * _K + b],
                                  wsem1).wait()

    return gather_kernel


def kernel(item_ids, table):
    batch, hist = item_ids.shape
    _, emb_dim = table.shape
    total = batch * hist
    assert total % (_NW * _CHUNK) == 0
    n_chunks = total // (_NW * _CHUNK)
    ids = item_ids.reshape(_NW, n_chunks, _CHUNK)
    out = _make_gather(n_chunks, emb_dim)(ids, table)
    return out.reshape(batch, hist, emb_dim)


# speedup vs baseline: 1.1137x; 1.1137x over previous
import functools

import jax
import jax.numpy as jnp
from jax import lax
from jax.experimental import pallas as pl
from jax.experimental.pallas import tpu as pltpu
from jax.experimental.pallas import tpu_sc as plsc

_NC = 2
_NS = 16
_NW = _NC * _NS
_CHUNK = 128
_K = 5


def _make_gather(n_chunks_total: int, hist: int, batch: int, emb_dim: int):
    cpt = n_chunks_total // _NW
    n_groups = cpt // _K
    assert n_chunks_total % _NW == 0 and cpt % _K == 0
    assert n_groups % 2 == 0 and n_groups >= 4
    mesh = plsc.VectorSubcoreMesh(core_axis_name="c", subcore_axis_name="s")

    @functools.partial(
        pl.kernel,
        out_type=jax.ShapeDtypeStruct((hist, emb_dim, batch), jnp.float32),
        mesh=mesh,
        compiler_params=pltpu.CompilerParams(use_tc_tiling_on_sc=False,
                                             needs_layout_passes=False),
        scratch_types=[
            pltpu.VMEM((cpt, _CHUNK), jnp.int32),
            pltpu.VMEM((2 * _K, _CHUNK, emb_dim), jnp.float32),
            pltpu.VMEM((2 * _K, emb_dim, _CHUNK), jnp.float32),
            pltpu.SemaphoreType.DMA,
            pltpu.SemaphoreType.DMA,
            pltpu.SemaphoreType.DMA,
            pltpu.SemaphoreType.DMA,
        ],
    )
    def gather_kernel(ids_hbm, table_hbm, out_hbm, idx_v, rows_v, trans_v,
                      gsem0, gsem1, wsem0, wsem1):
        wid = lax.axis_index("s") * _NC + lax.axis_index("c")
        gsems = (gsem0, gsem1)
        wsems = (wsem0, wsem1)
        c_base = wid * cpt

        def out_slice(c):
            h = lax.shift_right_logical(c, 7)
            b0 = pl.multiple_of(lax.shift_left(lax.bitwise_and(c, 127), 7),
                                _CHUNK)
            return out_hbm.at[h, :, pl.ds(b0, _CHUNK)]

        pltpu.sync_copy(ids_hbm.at[pl.ds(c_base, cpt)], idx_v)

        iotav = lax.iota(jnp.int32, 16)
        item_vecs = [iotav + (lg * 16) for lg in range(8)]

        def transpose_chunk(s):
            @pl.loop(0, emb_dim, unroll=4)
            def _(f):
                fvec = jnp.full((16,), 0, jnp.int32) + f
                for lg in range(8):
                    vals = plsc.load_gather(rows_v.at[s],
                                            [item_vecs[lg], fvec])
                    trans_v[s, f, pl.ds(lg * 16, 16)] = vals

        for b in range(_K):
            pltpu.async_copy(table_hbm.at[idx_v.at[b]], rows_v.at[b], gsem0)

        @pl.loop(0, n_groups, step=2)
        def _(g0):
            for hh in (0, 1):
                g = g0 + hh
                my = hh * _K
                other = (1 - hh) * _K
                @pl.when(jnp.logical_and(g >= 1, g + 1 < n_groups))
                def _():
                    for b in range(_K):
                        c = c_base + (g - 1) * _K + b
                        pltpu.make_async_copy(trans_v.at[other + b],
                                              out_slice(c),
                                              wsems[1 - hh]).wait()

                @pl.when(g + 1 < n_groups)
                def _():
                    for b in range(_K):
                        j = (g + 1) * _K + b
                        pltpu.async_copy(table_hbm.at[idx_v.at[j]],
                                         rows_v.at[other + b],
                                         gsems[1 - hh])

                for b in range(_K):
                    pltpu.make_async_copy(
                        table_hbm.at[idx_v.at[g * _K + b]],
                        rows_v.at[my + b], gsems[hh]).wait()
                for b in range(_K):
                    transpose_chunk(my + b)
                for b in range(_K):
                    c = c_base + g * _K + b
                    pltpu.async_copy(trans_v.at[my + b], out_slice(c),
                                     wsems[hh])

        for b in range(_K):
            c = c_base + (n_groups - 2) * _K + b
            pltpu.make_async_copy(trans_v.at[b], out_slice(c), wsem0).wait()
        for b in range(_K):
            c = c_base + (n_groups - 1) * _K + b
            pltpu.make_async_copy(trans_v.at[_K + b], out_slice(c),
                                  wsem1).wait()

    return gather_kernel


def kernel(item_ids, table):
    batch, hist = item_ids.shape
    _, emb_dim = table.shape
    total = batch * hist
    assert total % (_NW * _CHUNK) == 0 and batch % _CHUNK == 0
    n_chunks = total // _CHUNK
    ids = item_ids.T.reshape(n_chunks, _CHUNK)
    out = _make_gather(n_chunks, hist, batch, emb_dim)(ids, table)
    return out.transpose(2, 0, 1)


# speedup vs baseline: 1.4434x; 1.2960x over previous
import functools

import jax
import jax.numpy as jnp
from jax import lax
from jax.experimental import pallas as pl
from jax.experimental.pallas import tpu as pltpu
from jax.experimental.pallas import tpu_sc as plsc

_NC = 2
_NS = 16
_NW = _NC * _NS
_CHUNK = 128
_K = 5


def _make_gather(n_chunks_total: int, hist: int, batch: int, emb_dim: int):
    cpt = n_chunks_total // _NW
    n_groups = cpt // _K
    assert n_chunks_total % _NW == 0 and cpt % _K == 0
    assert n_groups % 2 == 0 and n_groups >= 4
    mesh = plsc.VectorSubcoreMesh(core_axis_name="c", subcore_axis_name="s")

    @functools.partial(
        pl.kernel,
        out_type=jax.ShapeDtypeStruct((hist, emb_dim, batch), jnp.float32),
        mesh=mesh,
        compiler_params=pltpu.CompilerParams(use_tc_tiling_on_sc=False,
                                             needs_layout_passes=False),
        scratch_types=[
            pltpu.VMEM((cpt, _CHUNK), jnp.int32),
            pltpu.VMEM((2 * _K, _CHUNK, emb_dim), jnp.float32),
            pltpu.VMEM((2 * _K, emb_dim, _CHUNK), jnp.float32),
            pltpu.SemaphoreType.DMA,
            pltpu.SemaphoreType.DMA,
            pltpu.SemaphoreType.DMA,
            pltpu.SemaphoreType.DMA,
        ],
    )
    def gather_kernel(ids_hbm, table_hbm, out_hbm, idx_v, rows_v, trans_v,
                      gsem0, gsem1, wsem0, wsem1):
        wid = lax.axis_index("s") * _NC + lax.axis_index("c")
        gsems = (gsem0, gsem1)
        wsems = (wsem0, wsem1)
        c_base = wid * cpt

        def out_slice(c):
            h = lax.shift_right_logical(c, 7)
            b0 = pl.multiple_of(lax.shift_left(lax.bitwise_and(c, 127), 7),
                                _CHUNK)
            return out_hbm.at[h, :, pl.ds(b0, _CHUNK)]

        pltpu.sync_copy(ids_hbm.at[pl.ds(c_base, cpt)], idx_v)

        iotav = lax.iota(jnp.int32, 16)
        item_vecs = [iotav + (lg * 16) for lg in range(8)]

        def transpose_chunk(s):
            @plsc.parallel_loop(0, emb_dim, unroll=4)
            def _(f):
                fvec = jnp.full((16,), 0, jnp.int32) + f
                for lg in range(8):
                    vals = plsc.load_gather(rows_v.at[s],
                                            [item_vecs[lg], fvec])
                    trans_v[s, f, pl.ds(lg * 16, 16)] = vals

        for b in range(_K):
            pltpu.async_copy(table_hbm.at[idx_v.at[b]], rows_v.at[b], gsem0)

        @pl.loop(0, n_groups, step=2)
        def _(g0):
            for hh in (0, 1):
                g = g0 + hh
                my = hh * _K
                other = (1 - hh) * _K
                @pl.when(jnp.logical_and(g >= 1, g + 1 < n_groups))
                def _():
                    for b in range(_K):
                        c = c_base + (g - 1) * _K + b
                        pltpu.make_async_copy(trans_v.at[other + b],
                                              out_slice(c),
                                              wsems[1 - hh]).wait()

                @pl.when(g + 1 < n_groups)
                def _():
                    for b in range(_K):
                        j = (g + 1) * _K + b
                        pltpu.async_copy(table_hbm.at[idx_v.at[j]],
                                         rows_v.at[other + b],
                                         gsems[1 - hh])

                for b in range(_K):
                    pltpu.make_async_copy(
                        table_hbm.at[idx_v.at[g * _K + b]],
                        rows_v.at[my + b], gsems[hh]).wait()
                for b in range(_K):
                    transpose_chunk(my + b)
                for b in range(_K):
                    c = c_base + g * _K + b
                    pltpu.async_copy(trans_v.at[my + b], out_slice(c),
                                     wsems[hh])

        for b in range(_K):
            c = c_base + (n_groups - 2) * _K + b
            pltpu.make_async_copy(trans_v.at[b], out_slice(c), wsem0).wait()
        for b in range(_K):
            c = c_base + (n_groups - 1) * _K + b
            pltpu.make_async_copy(trans_v.at[_K + b], out_slice(c),
                                  wsem1).wait()

    return gather_kernel


def kernel(item_ids, table):
    batch, hist = item_ids.shape
    _, emb_dim = table.shape
    total = batch * hist
    assert total % (_NW * _CHUNK) == 0 and batch % _CHUNK == 0
    n_chunks = total // _CHUNK
    ids = item_ids.T.reshape(n_chunks, _CHUNK)
    out = _make_gather(n_chunks, hist, batch, emb_dim)(ids, table)
    return out.transpose(2, 0, 1)


# speedup vs baseline: 1.4764x; 1.0228x over previous
import functools

import jax
import jax.numpy as jnp
from jax import lax
from jax.experimental import pallas as pl
from jax.experimental.pallas import tpu as pltpu
from jax.experimental.pallas import tpu_sc as plsc

_NC = 2
_NS = 16
_NW = _NC * _NS
_CHUNK = 128
_CPH = 4


def _make_gather(hist: int, batch: int, emb_dim: int):
    cols = _CPH * _CHUNK
    assert batch == _NW * cols
    mesh = plsc.VectorSubcoreMesh(core_axis_name="c", subcore_axis_name="s")

    @functools.partial(
        pl.kernel,
        out_type=jax.ShapeDtypeStruct((hist, emb_dim, batch), jnp.float32),
        mesh=mesh,
        compiler_params=pltpu.CompilerParams(use_tc_tiling_on_sc=False,
                                             needs_layout_passes=False),
        scratch_types=[
            pltpu.VMEM((hist, cols), jnp.int32),
            pltpu.VMEM((2 * _CPH, _CHUNK, emb_dim), jnp.float32),
            pltpu.VMEM((2, emb_dim, cols), jnp.float32),
            pltpu.SemaphoreType.DMA,
            pltpu.SemaphoreType.DMA,
            pltpu.SemaphoreType.DMA,
            pltpu.SemaphoreType.DMA,
        ],
    )
    def gather_kernel(ids_hbm, table_hbm, out_hbm, idx_v, rows_v, slab_v,
                      gsem0, gsem1, wsem0, wsem1):
        wid = lax.axis_index("s") * _NC + lax.axis_index("c")
        gsems = (gsem0, gsem1)
        wsems = (wsem0, wsem1)
        col0 = pl.multiple_of(wid * cols, cols)

        pltpu.sync_copy(ids_hbm.at[:, pl.ds(col0, cols)], idx_v)

        iotav = lax.iota(jnp.int32, 16)
        item_vecs = [iotav + (lg * 16) for lg in range(8)]

        def fire_gathers(h, half, sem):
            for cb in range(_CPH):
                pltpu.async_copy(
                    table_hbm.at[idx_v.at[h, pl.ds(cb * _CHUNK, _CHUNK)]],
                    rows_v.at[half * _CPH + cb], sem)

        def drain_gathers(h, half, sem):
            for cb in range(_CPH):
                pltpu.make_async_copy(
                    table_hbm.at[idx_v.at[h, pl.ds(cb * _CHUNK, _CHUNK)]],
                    rows_v.at[half * _CPH + cb], sem).wait()

        def transpose_group(half):
            @plsc.parallel_loop(0, emb_dim, unroll=2)
            def _(f):
                fvec = jnp.full((16,), 0, jnp.int32) + f
                for cb in range(_CPH):
                    for lg in range(8):
                        vals = plsc.load_gather(
                            rows_v.at[half * _CPH + cb],
                            [item_vecs[lg], fvec])
                        slab_v[half, f,
                               pl.ds(cb * _CHUNK + lg * 16, 16)] = vals

        def wb_copy(h, half, sem):
            return pltpu.make_async_copy(
                slab_v.at[half], out_hbm.at[h, :, pl.ds(col0, cols)], sem)

        fire_gathers(0, 0, gsem0)

        @pl.loop(0, hist, step=2)
        def _(h0):
            for hh in (0, 1):
                h = h0 + hh
                @pl.when(h >= 2)
                def _():
                    wb_copy(h - 2, hh, wsems[hh]).wait()

                @pl.when(h + 1 < hist)
                def _():
                    fire_gathers(h + 1, 1 - hh, gsems[1 - hh])

                drain_gathers(h, hh, gsems[hh])
                transpose_group(hh)
                pltpu.async_copy(slab_v.at[hh],
                                 out_hbm.at[h, :, pl.ds(col0, cols)],
                                 wsems[hh])

        wb_copy(hist - 2, 0, wsem0).wait()
        wb_copy(hist - 1, 1, wsem1).wait()

    return gather_kernel


def kernel(item_ids, table):
    batch, hist = item_ids.shape
    _, emb_dim = table.shape
    assert batch % (_NW * _CPH * _CHUNK) == 0 and hist % 2 == 0
    out = _make_gather(hist, batch, emb_dim)(item_ids.T, table)
    return out.transpose(2, 0, 1)
